# baseline (speedup 1.0000x reference)
import functools

import jax
import jax.numpy as jnp
from jax import lax
from jax.experimental import pallas as pl
from jax.experimental.pallas import tpu as pltpu
from jax.experimental.pallas import tpu_sc as plsc

_NUM_CORES = 2
_NUM_SUBCORES = 16
_LANES = 16
_NBUF = 3
_LOOKAHEAD = 2


def _embed_kernel(B, S, E, CW):
    NW = _NUM_CORES * _NUM_SUBCORES
    SP = S // NW
    NH = SP // CW

    mesh = plsc.VectorSubcoreMesh(core_axis_name="c", subcore_axis_name="s")

    scratch = [
        pltpu.VMEM((NH, B * CW), jnp.int32),
        pltpu.VMEM((SP, E), jnp.float32),
    ]
    scratch += [pltpu.VMEM((B * CW, E), jnp.float32) for _ in range(_NBUF)]
    scratch += [pltpu.SemaphoreType.DMA for _ in range(2 * _NBUF + 1 + NH)]

    @functools.partial(
        pl.kernel,
        mesh=mesh,
        out_type=jax.ShapeDtypeStruct((B, S, E), jnp.float32),
        scratch_types=scratch,
    )
    def k(x_hbm, tok_hbm, pos_hbm, out_hbm, idx_v, pos_v, *bufs):
        rows = bufs[0:_NBUF]
        sg = bufs[_NBUF:2 * _NBUF]
        so = bufs[2 * _NBUF:3 * _NBUF]
        s_idx = bufs[3 * _NBUF]
        s_pos = bufs[3 * _NBUF + 1:3 * _NBUF + 1 + NH]

        wid = lax.axis_index("s") * _NUM_CORES + lax.axis_index("c")
        p0 = wid * SP

        for h in range(NH):
            for b in range(B):
                pltpu.async_copy(
                    x_hbm.at[b, pl.ds(p0 + h * CW, CW)],
                    idx_v.at[h, pl.ds(b * CW, CW)],
                    s_idx,
                )
        for h in range(NH):
            pltpu.async_copy(
                pos_hbm.at[pl.ds(p0 + h * CW, CW)],
                pos_v.at[pl.ds(h * CW, CW)],
                s_pos[h],
            )
        for h in range(NH):
            for b in range(B):
                pltpu.make_async_copy(
                    x_hbm.at[b, pl.ds(p0 + h * CW, CW)],
                    idx_v.at[h, pl.ds(b * CW, CW)],
                    s_idx,
                ).wait()

        def wait_pos(h):
            pltpu.make_async_copy(
                pos_hbm.at[pl.ds(p0 + h * CW, CW)],
                pos_v.at[pl.ds(h * CW, CW)],
                s_pos[h],
            ).wait()

        def start(h):
            g = h % _NBUF
            pltpu.async_copy(tok_hbm.at[idx_v.at[h]], rows[g], sg[g])

        def wait_in(h):
            g = h % _NBUF
            pltpu.make_async_copy(
                tok_hbm.at[idx_v.at[h]], rows[g], sg[g]
            ).wait()

        def out_copy(h, b):
            g = h % _NBUF
            return pltpu.make_async_copy(
                rows[g].at[pl.ds(b * CW, CW)],
                out_hbm.at[b, pl.ds(p0 + h * CW, CW)],
                so[g],
            )

        def wait_out(h):
            for b in range(B):
                out_copy(h, b).wait()

        def add_and_store(h):
            g = h % _NBUF

            @pl.loop(0, CW)
            def _pos(p):
                @plsc.parallel_loop(0, E, step=_LANES, unroll=8)
                def _col(e):
                    pv = pos_v.at[h * CW + p, pl.ds(e, _LANES)][...]
                    for b in range(B):
                        plsc.addupdate(
                            rows[g].at[b * CW + p, pl.ds(e, _LANES)], pv
                        )

            for b in range(B):
                out_copy(h, b).start()

        for h in range(min(_LOOKAHEAD, NH)):
            start(h)
        for h in range(NH):
            wait_pos(h)
            wait_in(h)
            add_and_store(h)
            n = h + _LOOKAHEAD
            if n < NH:
                if n - _NBUF >= 0:
                    wait_out(n - _NBUF)
                start(n)
        for h in range(max(0, NH - _NBUF), NH):
            wait_out(h)

    return k


def kernel(x, tok_table, pos_table):
    B, S = x.shape
    _, E = tok_table.shape
    return _embed_kernel(B, S, E, CW=8)(
        x.astype(jnp.int32), tok_table, pos_table
    )


# speedup vs baseline: 1.0245x; 1.0245x over previous
import functools

import jax
import jax.numpy as jnp
from jax import lax
from jax.experimental import pallas as pl
from jax.experimental.pallas import tpu as pltpu
from jax.experimental.pallas import tpu_sc as plsc

_NUM_CORES = 2
_NUM_SUBCORES = 16
_LANES = 16
_NBUF = 4
_LOOKAHEAD = 3


def _embed_kernel(B, S, E, CW):
    NW = _NUM_CORES * _NUM_SUBCORES
    SP = S // NW
    NH = SP // CW

    mesh = plsc.VectorSubcoreMesh(core_axis_name="c", subcore_axis_name="s")

    scratch = [
        pltpu.VMEM((NH, B * CW), jnp.int32),
        pltpu.VMEM((_NBUF, CW, E), jnp.float32),
    ]
    scratch += [pltpu.VMEM((B * CW, E), jnp.float32) for _ in range(_NBUF)]
    scratch += [pltpu.SemaphoreType.DMA for _ in range(3 * _NBUF + 1)]

    @functools.partial(
        pl.kernel,
        mesh=mesh,
        out_type=jax.ShapeDtypeStruct((B, S, E), jnp.float32),
        scratch_types=scratch,
    )
    def k(x_hbm, tok_hbm, pos_hbm, out_hbm, idx_v, pos_v, *bufs):
        rows = bufs[0:_NBUF]
        sg = bufs[_NBUF:2 * _NBUF]
        so = bufs[2 * _NBUF:3 * _NBUF]
        s_idx = bufs[3 * _NBUF]
        s_pos = bufs[3 * _NBUF + 1:4 * _NBUF + 1]

        wid = lax.axis_index("s") * _NUM_CORES + lax.axis_index("c")
        p0 = wid * SP

        for h in range(NH):
            for b in range(B):
                pltpu.async_copy(
                    x_hbm.at[b, pl.ds(p0 + h * CW, CW)],
                    idx_v.at[h, pl.ds(b * CW, CW)],
                    s_idx,
                )
        for h in range(NH):
            for b in range(B):
                pltpu.make_async_copy(
                    x_hbm.at[b, pl.ds(p0 + h * CW, CW)],
                    idx_v.at[h, pl.ds(b * CW, CW)],
                    s_idx,
                ).wait()

        def pos_copy(h):
            g = h % _NBUF
            return pltpu.make_async_copy(
                pos_hbm.at[pl.ds(p0 + h * CW, CW)], pos_v.at[g], s_pos[g]
            )

        def start(h):
            g = h % _NBUF
            pltpu.async_copy(tok_hbm.at[idx_v.at[h]], rows[g], sg[g])

        def wait_in(h):
            g = h % _NBUF
            pltpu.make_async_copy(
                tok_hbm.at[idx_v.at[h]], rows[g], sg[g]
            ).wait()

        def out_copy(h, b):
            g = h % _NBUF
            return pltpu.make_async_copy(
                rows[g].at[pl.ds(b * CW, CW)],
                out_hbm.at[b, pl.ds(p0 + h * CW, CW)],
                so[g],
            )

        def wait_out(h):
            for b in range(B):
                out_copy(h, b).wait()

        def add_and_store(h):
            g = h % _NBUF

            @pl.loop(0, CW)
            def _pos(p):
                @plsc.parallel_loop(0, E, step=_LANES, unroll=4)
                def _col(e):
                    pv = pos_v.at[h % _NBUF, p, pl.ds(e, _LANES)][...]
                    for b in range(B):
                        plsc.addupdate(
                            rows[g].at[b * CW + p, pl.ds(e, _LANES)], pv
                        )

            for b in range(B):
                out_copy(h, b).start()

        for h in range(min(_LOOKAHEAD, NH)):
            start(h)
            pos_copy(h).start()
        for h in range(NH):
            pos_copy(h).wait()
            wait_in(h)
            add_and_store(h)
            n = h + _LOOKAHEAD
            if n < NH:
                if n - _NBUF >= 0:
                    wait_out(n - _NBUF)
                start(n)
                pos_copy(n).start()
        for h in range(max(0, NH - _NBUF), NH):
            wait_out(h)

    return k


def kernel(x, tok_table, pos_table):
    B, S = x.shape
    _, E = tok_table.shape
    return _embed_kernel(B, S, E, CW=8)(
        x.astype(jnp.int32), tok_table, pos_table
    )
